# baseline (speedup 1.0000x reference)
import functools

import jax
import jax.numpy as jnp
from jax import lax
from jax.experimental import pallas as pl
from jax.experimental.pallas import tpu as pltpu
from jax.experimental.pallas import tpu_sc as plsc

N = 10000
D = 128
E = 160000
LEAKY = 0.2

NP = 10240
BN = 1024
NSUB = 16
ET = E // NSUB
KC = 32
NFULL = ET // KC
NTRI = NFULL // 3
TAIL = ET - NFULL * KC
RT = NP // NSUB
NRB = RT // KC


def _prep_body(h_ref, w0_ref, w1_ref, a0_ref, a1_ref,
               wh0_ref, wh1_ref, sd_ref, ss_ref, cm_ref):
  h = h_ref[...]
  wh0 = jnp.dot(h, w0_ref[...].T, preferred_element_type=jnp.float32)
  wh1 = jnp.dot(h, w1_ref[...].T, preferred_element_type=jnp.float32)
  wh0_ref[...] = wh0
  wh1_ref[...] = wh1
  a0 = a0_ref[...]
  a1 = a1_ref[...]
  s0d = wh0 @ a0[0]
  s0s = wh0 @ a0[1]
  s1d = wh1 @ a1[0]
  s1s = wh1 @ a1[1]
  sd_ref[...] = jnp.stack([s0d, s1d], axis=0)
  ss_ref[...] = jnp.stack([s0s, s1s], axis=0)
  vals = jnp.concatenate([
      jnp.full((1, D), jnp.max(s0d), jnp.float32),
      jnp.full((1, D), jnp.max(s0s), jnp.float32),
      jnp.full((1, D), jnp.max(s1d), jnp.float32),
      jnp.full((1, D), jnp.max(s1s), jnp.float32),
      jnp.full((4, D), -3e38, jnp.float32),
  ], axis=0)
  j = pl.program_id(0)

  @pl.when(j == 0)
  def _():
    cm_ref[...] = vals

  @pl.when(j > 0)
  def _():
    cm_ref[...] = jnp.maximum(cm_ref[...], vals)


def _prep(h_pad, w0, w1, a0, a1):
  grid = NP // BN
  return pl.pallas_call(
      _prep_body,
      grid=(grid,),
      in_specs=[
          pl.BlockSpec((BN, D), lambda j: (j, 0)),
          pl.BlockSpec((D, D), lambda j: (0, 0)),
          pl.BlockSpec((D, D), lambda j: (0, 0)),
          pl.BlockSpec((2, D), lambda j: (0, 0)),
          pl.BlockSpec((2, D), lambda j: (0, 0)),
      ],
      out_specs=[
          pl.BlockSpec((BN, D), lambda j: (j, 0)),
          pl.BlockSpec((BN, D), lambda j: (j, 0)),
          pl.BlockSpec((2, BN), lambda j: (0, j)),
          pl.BlockSpec((2, BN), lambda j: (0, j)),
          pl.BlockSpec((8, D), lambda j: (0, 0)),
      ],
      out_shape=[
          jax.ShapeDtypeStruct((NP, D), jnp.float32),
          jax.ShapeDtypeStruct((NP, D), jnp.float32),
          jax.ShapeDtypeStruct((2, NP), jnp.float32),
          jax.ShapeDtypeStruct((2, NP), jnp.float32),
          jax.ShapeDtypeStruct((8, D), jnp.float32),
      ],
  )(h_pad, w0, w1, a0, a1)


def _sc_gat(wh0, wh1, sd0, ss0, sd1, ss1, row0, col0, row1, col1, cm):
  mesh = plsc.VectorSubcoreMesh(core_axis_name="c", subcore_axis_name="s")

  @functools.partial(
      pl.kernel,
      out_type=(
          jax.ShapeDtypeStruct((NP, D), jnp.float32),
          jax.ShapeDtypeStruct((NP, D), jnp.float32),
          jax.ShapeDtypeStruct((NSUB * NP,), jnp.float32),
          jax.ShapeDtypeStruct((NSUB * NP,), jnp.float32),
      ),
      mesh=mesh,
      compiler_params=pltpu.CompilerParams(needs_layout_passes=False),
      scratch_types=(
          pltpu.VMEM((N,), jnp.float32),
          pltpu.VMEM((N,), jnp.float32),
          pltpu.VMEM((KC,), jnp.int32),
          pltpu.VMEM((KC,), jnp.int32),
          pltpu.VMEM((KC, D), jnp.float32),
          pltpu.VMEM((KC,), jnp.int32),
          pltpu.VMEM((KC,), jnp.int32),
          pltpu.VMEM((KC,), jnp.int32),
          pltpu.VMEM((KC, D), jnp.float32),
          pltpu.VMEM((KC,), jnp.int32),
          pltpu.VMEM((KC,), jnp.int32),
          pltpu.VMEM((KC,), jnp.int32),
          pltpu.VMEM((KC, D), jnp.float32),
          pltpu.VMEM((KC,), jnp.int32),
          pltpu.VMEM((16,), jnp.int32),
          pltpu.VMEM((16,), jnp.int32),
          pltpu.VMEM((16,), jnp.float32),
          pltpu.VMEM((16,), jnp.float32),
          pltpu.VMEM((N,), jnp.float32),
          pltpu.VMEM((RT,), jnp.float32),
          pltpu.VMEM((RT - 128,), jnp.float32),
          pltpu.VMEM_SHARED((NP, D), jnp.float32),
          pltpu.SemaphoreType.DMA,
          pltpu.SemaphoreType.DMA,
          pltpu.SemaphoreType.DMA,
          pltpu.SemaphoreType.DMA,
          pltpu.SemaphoreType.DMA,
          pltpu.SemaphoreType.DMA,
          pltpu.SemaphoreType.DMA,
          pltpu.SemaphoreType.DMA,
          pltpu.SemaphoreType.DMA,
      ),
  )
  def k(wh0_h, wh1_h, sd0_h, ss0_h, sd1_h, ss1_h,
        row0_h, col0_h, row1_h, col1_h, cm_h,
        out0_h, out1_h, den0_h, den1_h,
        sd_v, ss_v, ir0, ic0, rg0, is0, ir1, ic1, rg1, is1,
        ir2, ic2, rg2, is2,
        irt, ict, cd_v, cs_v, den_v, dtot_v, dpart_v, agg,
        sg0, sg1, sg2, si0, si1, si2, sc0, sc1, sc2):
    c = lax.axis_index("c")
    s = lax.axis_index("s")
    zeros16 = jnp.zeros((16,), jnp.float32)
    bufs = ((ir0, ic0, rg0, is0, sg0, si0, sc0),
            (ir1, ic1, rg1, is1, sg1, si1, sc1),
            (ir2, ic2, rg2, is2, sg2, si2, sc2))

    def phase1(sd_h, ss_h):
      pltpu.sync_copy(sd_h.at[pl.ds(0, N)], sd_v)
      pltpu.sync_copy(ss_h.at[pl.ds(0, N)], ss_v)

      def zrow(i, _):
        for b in range(D // 16):
          rg0[i, pl.ds(b * 16, 16)] = zeros16
        return 0
      lax.fori_loop(0, KC, zrow, 0)
      for t in range(NRB):
        pltpu.sync_copy(rg0, agg.at[pl.ds(s * RT + t * KC, KC), :])

      def zden(i, _):
        den_v[pl.ds(i * 16, 16)] = zeros16
        return 0
      lax.fori_loop(0, N // 16, zden, 0)

    def phase2(row_h, col_h, wh_h, den_h, crow):
      pltpu.sync_copy(cm_h.at[crow, pl.ds(0, 16)], cd_v)
      pltpu.sync_copy(cm_h.at[crow + 1, pl.ds(0, 16)], cs_v)
      cc = cd_v[...] + cs_v[...]
      tile_base = s * ET

      def start_idx(g, b):
        eb = tile_base + g * KC
        ir_b, ic_b, _, _, _, si_b, _ = bufs[b]
        pltpu.async_copy(row_h.at[pl.ds(eb, KC)], ir_b, si_b)
        pltpu.async_copy(col_h.at[pl.ds(eb, KC)], ic_b, si_b)

      def wait_idx(b):
        ir_b, ic_b, _, _, _, si_b, _ = bufs[b]
        pltpu.make_async_copy(row_h.at[pl.ds(0, KC)], ir_b, si_b).wait()
        pltpu.make_async_copy(col_h.at[pl.ds(0, KC)], ic_b, si_b).wait()

      def start_gather(b):
        _, ic_b, rg_b, _, sg_b, _, _ = bufs[b]
        pltpu.async_copy(wh_h.at[ic_b], rg_b, sg_b)

      def wait_gather(b):
        _, ic_b, rg_b, _, sg_b, _, _ = bufs[b]
        pltpu.make_async_copy(wh_h.at[ic_b], rg_b, sg_b).wait()

      def start_scatter(b):
        _, _, rg_b, is_b, _, _, sc_b = bufs[b]
        pltpu.async_copy(rg_b, agg.at[is_b], sc_b, add=True)

      def wait_scatter(b):
        _, _, rg_b, is_b, _, _, sc_b = bufs[b]
        pltpu.make_async_copy(rg_b, agg.at[is_b], sc_b).wait()


      def edge_block(ir_v, ic_v, rows_ref, j):
        ir = ir_v[pl.ds(j * 16, 16)]
        ic = ic_v[pl.ds(j * 16, 16)]
        e = plsc.load_gather(sd_v, [ir]) + plsc.load_gather(ss_v, [ic])
        e = jnp.where(e >= 0.0, e, e * LEAKY)
        p = jnp.exp(e - cc)
        plsc.addupdate_scatter(den_v, [ir], p)
        for lane in range(16):
          i = j * 16 + lane
          p_i = p[lane]
          for bb in range(D // 16):
            rows_ref[i, pl.ds(bb * 16, 16)] = (
                rows_ref[i, pl.ds(bb * 16, 16)] * p_i)

      def compute(b):
        ir_b, ic_b, rg_b, is_b, _, _, _ = bufs[b]

        def rblk(j, _):
          edge_block(ir_b, ic_b, rg_b, j)
          return 0
        lax.fori_loop(0, KC // 16, rblk, 0)
        for j in range(KC // 16):
          is_b[pl.ds(j * 16, 16)] = ir_b[pl.ds(j * 16, 16)]

      start_idx(0, 0)
      start_idx(1, 1)
      start_idx(2, 2)
      wait_idx(0)
      start_gather(0)
      wait_idx(1)
      start_gather(1)

      def triple(t, _):
        for kk in range(3):
          b = kk
          b1 = (kk + 1) % 3
          b2 = (kk + 2) % 3
          g = 3 * t + kk
          wait_gather(b)
          compute(b)
          start_scatter(b)
          if kk == 0:
            pl.when(t > 0)(lambda: wait_scatter(b2))
          else:
            wait_scatter(b2)
          if kk == 0:
            wait_idx(b2)
            start_gather(b2)
            pl.when(t < NTRI - 1)(lambda: start_idx(g + 3, b))
          else:
            @pl.when(t < NTRI - 1)
            def _(b=b, b2=b2, g=g):
              wait_idx(b2)
              start_gather(b2)
              start_idx(g + 3, b)
        return 0
      lax.fori_loop(0, NTRI, triple, 0)
      wait_scatter(2)

      tb = tile_base + NFULL * KC
      pltpu.sync_copy(row_h.at[pl.ds(tb, TAIL)], irt)
      pltpu.sync_copy(col_h.at[pl.ds(tb, TAIL)], ict)
      pltpu.async_copy(wh_h.at[ict], rg0.at[pl.ds(0, TAIL), :], sg0).wait()
      edge_block(irt, ict, rg0, 0)
      pltpu.sync_copy(rg0.at[pl.ds(0, TAIL), :], agg.at[irt], add=True)

      pltpu.sync_copy(den_v, den_h.at[pl.ds(s * NP, N)])

    def phase3(out_h, den_h):
      pltpu.sync_copy(den_h.at[pl.ds(s * RT, RT)], dtot_v)
      for t in range(1, NSUB):
        for (off, ln) in ((0, RT - 128), (RT - 128, 128)):
          pltpu.sync_copy(den_h.at[pl.ds(t * NP + s * RT + off, ln)],
                          dpart_v.at[pl.ds(0, ln)])

          def dacc(i, _):
            dtot_v[pl.ds(off + i * 16, 16)] = (
                dtot_v[pl.ds(off + i * 16, 16)] + dpart_v[pl.ds(i * 16, 16)])
            return 0
          lax.fori_loop(0, ln // 16, dacc, 0)

      def dinv(i, _):
        d = dtot_v[pl.ds(i * 16, 16)]
        dtot_v[pl.ds(i * 16, 16)] = 1.0 / jnp.maximum(d, 1e-12)
        return 0
      lax.fori_loop(0, RT // 16, dinv, 0)

      def blk(t, _):
        base = s * RT + t * KC
        pltpu.sync_copy(agg.at[pl.ds(base, KC), :], rg0)

        def nblk(j, _):
          iv16 = dtot_v[pl.ds(t * KC + j * 16, 16)]
          for lane in range(16):
            i = j * 16 + lane
            inv = iv16[lane]
            for b in range(D // 16):
              rg0[i, pl.ds(b * 16, 16)] = (
                  rg0[i, pl.ds(b * 16, 16)] * inv)
          return 0
        lax.fori_loop(0, KC // 16, nblk, 0)
        pltpu.sync_copy(rg0, out_h.at[pl.ds(base, KC), :])
        return 0
      lax.fori_loop(0, NRB, blk, 0)

    pl.when(c == 0)(lambda: phase1(sd0_h, ss0_h))
    pl.when(c == 1)(lambda: phase1(sd1_h, ss1_h))
    plsc.subcore_barrier()
    pl.when(c == 0)(lambda: phase2(row0_h, col0_h, wh0_h, den0_h, 0))
    pl.when(c == 1)(lambda: phase2(row1_h, col1_h, wh1_h, den1_h, 2))
    plsc.subcore_barrier()
    pl.when(c == 0)(lambda: phase3(out0_h, den0_h))
    pl.when(c == 1)(lambda: phase3(out1_h, den1_h))

  return k(wh0, wh1, sd0, ss0, sd1, ss1, row0, col0, row1, col1, cm)


def _combine_body(c0_ref, c1_ref, b_ref, o_ref):
  o_ref[...] = c0_ref[...] + c1_ref[...] + b_ref[...]


def _combine(c0, c1, bias2d):
  grid = NP // BN
  return pl.pallas_call(
      _combine_body,
      grid=(grid,),
      in_specs=[
          pl.BlockSpec((BN, D), lambda j: (j, 0)),
          pl.BlockSpec((BN, D), lambda j: (j, 0)),
          pl.BlockSpec((1, D), lambda j: (0, 0)),
      ],
      out_specs=pl.BlockSpec((BN, D), lambda j: (j, 0)),
      out_shape=jax.ShapeDtypeStruct((NP, D), jnp.float32),
  )(c0, c1, bias2d)


@jax.jit
def kernel(H, W_r0, W_r1, a_r0, a_r1, bias, row_r0, col_r0, row_r1, col_r1):
  h_pad = jnp.pad(H, ((0, NP - N), (0, 0)))
  a0 = a_r0.reshape(2, D)
  a1 = a_r1.reshape(2, D)
  wh0, wh1, sd, ss, cm = _prep(h_pad, W_r0, W_r1, a0, a1)
  c0, c1, _, _ = _sc_gat(
      wh0, wh1, sd[0], ss[0], sd[1], ss[1],
      row_r0.astype(jnp.int32), col_r0.astype(jnp.int32),
      row_r1.astype(jnp.int32), col_r1.astype(jnp.int32), cm,
  )
  out = _combine(c0, c1, bias.reshape(1, D))
  return out[:N]


# speedup vs baseline: 1.0547x; 1.0547x over previous
import functools

import jax
import jax.numpy as jnp
from jax import lax
from jax.experimental import pallas as pl
from jax.experimental.pallas import tpu as pltpu
from jax.experimental.pallas import tpu_sc as plsc

N = 10000
D = 128
E = 160000
LEAKY = 0.2

NP = 10240
BN = 1024
NSUB = 16
ET = E // NSUB
KC = 64
NFULL = ET // KC
NPAIR = NFULL // 2
TAIL = ET - NFULL * KC
RT = NP // NSUB
NRB = RT // KC


def _prep_body(h_ref, w0_ref, w1_ref, a0_ref, a1_ref,
               wh0_ref, wh1_ref, sd_ref, ss_ref, cm_ref):
  h = h_ref[...]
  wh0 = jnp.dot(h, w0_ref[...].T, preferred_element_type=jnp.float32)
  wh1 = jnp.dot(h, w1_ref[...].T, preferred_element_type=jnp.float32)
  wh0_ref[...] = wh0
  wh1_ref[...] = wh1
  a0 = a0_ref[...]
  a1 = a1_ref[...]
  s0d = wh0 @ a0[0]
  s0s = wh0 @ a0[1]
  s1d = wh1 @ a1[0]
  s1s = wh1 @ a1[1]
  sd_ref[...] = jnp.stack([s0d, s1d], axis=0)
  ss_ref[...] = jnp.stack([s0s, s1s], axis=0)
  vals = jnp.concatenate([
      jnp.full((1, D), jnp.max(s0d), jnp.float32),
      jnp.full((1, D), jnp.max(s0s), jnp.float32),
      jnp.full((1, D), jnp.max(s1d), jnp.float32),
      jnp.full((1, D), jnp.max(s1s), jnp.float32),
      jnp.full((4, D), -3e38, jnp.float32),
  ], axis=0)
  j = pl.program_id(0)

  @pl.when(j == 0)
  def _():
    cm_ref[...] = vals

  @pl.when(j > 0)
  def _():
    cm_ref[...] = jnp.maximum(cm_ref[...], vals)


def _prep(h_pad, w0, w1, a0, a1):
  grid = NP // BN
  return pl.pallas_call(
      _prep_body,
      grid=(grid,),
      in_specs=[
          pl.BlockSpec((BN, D), lambda j: (j, 0)),
          pl.BlockSpec((D, D), lambda j: (0, 0)),
          pl.BlockSpec((D, D), lambda j: (0, 0)),
          pl.BlockSpec((2, D), lambda j: (0, 0)),
          pl.BlockSpec((2, D), lambda j: (0, 0)),
      ],
      out_specs=[
          pl.BlockSpec((BN, D), lambda j: (j, 0)),
          pl.BlockSpec((BN, D), lambda j: (j, 0)),
          pl.BlockSpec((2, BN), lambda j: (0, j)),
          pl.BlockSpec((2, BN), lambda j: (0, j)),
          pl.BlockSpec((8, D), lambda j: (0, 0)),
      ],
      out_shape=[
          jax.ShapeDtypeStruct((NP, D), jnp.float32),
          jax.ShapeDtypeStruct((NP, D), jnp.float32),
          jax.ShapeDtypeStruct((2, NP), jnp.float32),
          jax.ShapeDtypeStruct((2, NP), jnp.float32),
          jax.ShapeDtypeStruct((8, D), jnp.float32),
      ],
  )(h_pad, w0, w1, a0, a1)


def _sc_gat(wh0, wh1, sd0, ss0, sd1, ss1, row0, col0, row1, col1, cm):
  mesh = plsc.VectorSubcoreMesh(core_axis_name="c", subcore_axis_name="s")

  @functools.partial(
      pl.kernel,
      out_type=(
          jax.ShapeDtypeStruct((NP, D), jnp.float32),
          jax.ShapeDtypeStruct((NP, D), jnp.float32),
          jax.ShapeDtypeStruct((NSUB * NP,), jnp.float32),
          jax.ShapeDtypeStruct((NSUB * NP,), jnp.float32),
      ),
      mesh=mesh,
      compiler_params=pltpu.CompilerParams(needs_layout_passes=False),
      scratch_types=(
          pltpu.VMEM((N,), jnp.float32),
          pltpu.VMEM((N,), jnp.float32),
          pltpu.VMEM((KC,), jnp.int32),
          pltpu.VMEM((KC,), jnp.int32),
          pltpu.VMEM((KC, D), jnp.float32),
          pltpu.VMEM((KC,), jnp.int32),
          pltpu.VMEM((KC,), jnp.int32),
          pltpu.VMEM((KC,), jnp.int32),
          pltpu.VMEM((KC, D), jnp.float32),
          pltpu.VMEM((KC,), jnp.int32),
          pltpu.VMEM((16,), jnp.int32),
          pltpu.VMEM((16,), jnp.int32),
          pltpu.VMEM((16,), jnp.float32),
          pltpu.VMEM((16,), jnp.float32),
          pltpu.VMEM((N,), jnp.float32),
          pltpu.VMEM((RT,), jnp.float32),
          pltpu.VMEM((RT - 128,), jnp.float32),
          pltpu.VMEM_SHARED((NP, D), jnp.float32),
          pltpu.SemaphoreType.DMA,
          pltpu.SemaphoreType.DMA,
          pltpu.SemaphoreType.DMA,
          pltpu.SemaphoreType.DMA,
      ),
  )
  def k(wh0_h, wh1_h, sd0_h, ss0_h, sd1_h, ss1_h,
        row0_h, col0_h, row1_h, col1_h, cm_h,
        out0_h, out1_h, den0_h, den1_h,
        sd_v, ss_v, ir0, ic0, rg0, is0, ir1, ic1, rg1, is1,
        irt, ict, cd_v, cs_v, den_v, dtot_v, dpart_v, agg,
        sg0, sg1, si0, si1):
    c = lax.axis_index("c")
    s = lax.axis_index("s")
    zeros16 = jnp.zeros((16,), jnp.float32)
    bufs = ((ir0, ic0, rg0, is0, sg0, si0), (ir1, ic1, rg1, is1, sg1, si1))

    def phase1(sd_h, ss_h):
      pltpu.sync_copy(sd_h.at[pl.ds(0, N)], sd_v)
      pltpu.sync_copy(ss_h.at[pl.ds(0, N)], ss_v)

      def zrow(i, _):
        for b in range(D // 16):
          rg0[i, pl.ds(b * 16, 16)] = zeros16
        return 0
      lax.fori_loop(0, KC, zrow, 0)
      for t in range(NRB):
        pltpu.sync_copy(rg0, agg.at[pl.ds(s * RT + t * KC, KC), :])

      def zden(i, _):
        den_v[pl.ds(i * 16, 16)] = zeros16
        return 0
      lax.fori_loop(0, N // 16, zden, 0)

    def phase2(row_h, col_h, wh_h, den_h, crow):
      pltpu.sync_copy(cm_h.at[crow, pl.ds(0, 16)], cd_v)
      pltpu.sync_copy(cm_h.at[crow + 1, pl.ds(0, 16)], cs_v)
      cc = cd_v[...] + cs_v[...]
      tile_base = s * ET

      def start_idx(g, b):
        eb = tile_base + g * KC
        ir_b, ic_b, _, _, _, si_b = bufs[b]
        pltpu.async_copy(row_h.at[pl.ds(eb, KC)], ir_b, si_b)
        pltpu.async_copy(col_h.at[pl.ds(eb, KC)], ic_b, si_b)

      def wait_idx(b):
        ir_b, ic_b, _, _, _, si_b = bufs[b]
        pltpu.make_async_copy(row_h.at[pl.ds(0, KC)], ir_b, si_b).wait()
        pltpu.make_async_copy(col_h.at[pl.ds(0, KC)], ic_b, si_b).wait()

      def start_gather(b):
        _, ic_b, rg_b, _, sg_b, _ = bufs[b]
        pltpu.async_copy(wh_h.at[ic_b], rg_b, sg_b)

      def wait_gather(b):
        _, ic_b, rg_b, _, sg_b, _ = bufs[b]
        pltpu.make_async_copy(wh_h.at[ic_b], rg_b, sg_b).wait()

      def scatter(b):
        _, _, rg_b, is_b, _, _ = bufs[b]
        pltpu.sync_copy(rg_b, agg.at[is_b], add=True)


      def edge_block(ir_v, ic_v, rows_ref, j):
        ir = ir_v[pl.ds(j * 16, 16)]
        ic = ic_v[pl.ds(j * 16, 16)]
        e = plsc.load_gather(sd_v, [ir]) + plsc.load_gather(ss_v, [ic])
        e = jnp.where(e >= 0.0, e, e * LEAKY)
        p = jnp.exp(e - cc)
        plsc.addupdate_scatter(den_v, [ir], p)
        for lane in range(16):
          i = j * 16 + lane
          p_i = p[lane]
          for bb in range(D // 16):
            rows_ref[i, pl.ds(bb * 16, 16)] = (
                rows_ref[i, pl.ds(bb * 16, 16)] * p_i)

      def compute(b):
        ir_b, ic_b, rg_b, is_b, _, _ = bufs[b]

        def rblk(j, _):
          edge_block(ir_b, ic_b, rg_b, j)
          return 0
        lax.fori_loop(0, KC // 16, rblk, 0)
        for j in range(KC // 16):
          is_b[pl.ds(j * 16, 16)] = ir_b[pl.ds(j * 16, 16)]

      start_idx(0, 0)
      wait_idx(0)
      start_gather(0)
      start_idx(1, 1)

      def pair(t, _):
        wait_idx(1)
        start_gather(1)
        wait_gather(0)
        compute(0)

        @pl.when(t < NPAIR - 1)
        def _():
          start_idx(2 * t + 2, 0)
        scatter(0)

        @pl.when(t < NPAIR - 1)
        def _():
          wait_idx(0)
          start_gather(0)
        wait_gather(1)
        compute(1)

        @pl.when(t < NPAIR - 1)
        def _():
          start_idx(2 * t + 3, 1)
        scatter(1)
        return 0
      lax.fori_loop(0, NPAIR, pair, 0)

      tb = tile_base + NFULL * KC
      pltpu.sync_copy(row_h.at[pl.ds(tb, TAIL)], irt)
      pltpu.sync_copy(col_h.at[pl.ds(tb, TAIL)], ict)
      pltpu.async_copy(wh_h.at[ict], rg0.at[pl.ds(0, TAIL), :], sg0).wait()
      edge_block(irt, ict, rg0, 0)
      pltpu.sync_copy(rg0.at[pl.ds(0, TAIL), :], agg.at[irt], add=True)

      pltpu.sync_copy(den_v, den_h.at[pl.ds(s * NP, N)])

    def phase3(out_h, den_h):
      pltpu.sync_copy(den_h.at[pl.ds(s * RT, RT)], dtot_v)
      for t in range(1, NSUB):
        for (off, ln) in ((0, RT - 128), (RT - 128, 128)):
          pltpu.sync_copy(den_h.at[pl.ds(t * NP + s * RT + off, ln)],
                          dpart_v.at[pl.ds(0, ln)])

          def dacc(i, _):
            dtot_v[pl.ds(off + i * 16, 16)] = (
                dtot_v[pl.ds(off + i * 16, 16)] + dpart_v[pl.ds(i * 16, 16)])
            return 0
          lax.fori_loop(0, ln // 16, dacc, 0)

      def dinv(i, _):
        d = dtot_v[pl.ds(i * 16, 16)]
        dtot_v[pl.ds(i * 16, 16)] = 1.0 / jnp.maximum(d, 1e-12)
        return 0
      lax.fori_loop(0, RT // 16, dinv, 0)

      def blk(t, _):
        base = s * RT + t * KC
        pltpu.sync_copy(agg.at[pl.ds(base, KC), :], rg0)

        def nblk(j, _):
          iv16 = dtot_v[pl.ds(t * KC + j * 16, 16)]
          for lane in range(16):
            i = j * 16 + lane
            inv = iv16[lane]
            for b in range(D // 16):
              rg0[i, pl.ds(b * 16, 16)] = (
                  rg0[i, pl.ds(b * 16, 16)] * inv)
          return 0
        lax.fori_loop(0, KC // 16, nblk, 0)
        pltpu.sync_copy(rg0, out_h.at[pl.ds(base, KC), :])
        return 0
      lax.fori_loop(0, NRB, blk, 0)

    pl.when(c == 0)(lambda: phase1(sd0_h, ss0_h))
    pl.when(c == 1)(lambda: phase1(sd1_h, ss1_h))
    plsc.subcore_barrier()
    pl.when(c == 0)(lambda: phase2(row0_h, col0_h, wh0_h, den0_h, 0))
    pl.when(c == 1)(lambda: phase2(row1_h, col1_h, wh1_h, den1_h, 2))
    plsc.subcore_barrier()
    pl.when(c == 0)(lambda: phase3(out0_h, den0_h))
    pl.when(c == 1)(lambda: phase3(out1_h, den1_h))

  return k(wh0, wh1, sd0, ss0, sd1, ss1, row0, col0, row1, col1, cm)


def _combine_body(c0_ref, c1_ref, b_ref, o_ref):
  o_ref[...] = c0_ref[...] + c1_ref[...] + b_ref[...]


def _combine(c0, c1, bias2d):
  bc = 400
  return pl.pallas_call(
      _combine_body,
      grid=(N // bc,),
      in_specs=[
          pl.BlockSpec((bc, D), lambda j: (j, 0)),
          pl.BlockSpec((bc, D), lambda j: (j, 0)),
          pl.BlockSpec((1, D), lambda j: (0, 0)),
      ],
      out_specs=pl.BlockSpec((bc, D), lambda j: (j, 0)),
      out_shape=jax.ShapeDtypeStruct((N, D), jnp.float32),
  )(c0, c1, bias2d)


@jax.jit
def kernel(H, W_r0, W_r1, a_r0, a_r1, bias, row_r0, col_r0, row_r1, col_r1):
  h_pad = jnp.pad(H, ((0, NP - N), (0, 0)))
  a0 = a_r0.reshape(2, D)
  a1 = a_r1.reshape(2, D)
  wh0, wh1, sd, ss, cm = _prep(h_pad, W_r0, W_r1, a0, a1)
  c0, c1, _, _ = _sc_gat(
      wh0, wh1, sd[0], ss[0], sd[1], ss[1],
      row_r0.astype(jnp.int32), col_r0.astype(jnp.int32),
      row_r1.astype(jnp.int32), col_r1.astype(jnp.int32), cm,
  )
  return _combine(c0, c1, bias.reshape(1, D))


# speedup vs baseline: 1.0656x; 1.0103x over previous
import functools

import jax
import jax.numpy as jnp
from jax import lax
from jax.experimental import pallas as pl
from jax.experimental.pallas import tpu as pltpu
from jax.experimental.pallas import tpu_sc as plsc

N = 10000
D = 128
E = 160000
LEAKY = 0.2

NP = 10240
BN = 1024
NSUB = 16
ET = E // NSUB
KC = 64
NFULL = ET // KC
NPAIR = NFULL // 2
TAIL = ET - NFULL * KC
RT = NP // NSUB
NRB = RT // KC


def _prep_body(h_ref, w0_ref, w1_ref, a0_ref, a1_ref,
               wh0_ref, wh1_ref, sd_ref, ss_ref, cm_ref):
  h = h_ref[...]
  wh0 = jnp.dot(h, w0_ref[...].T, preferred_element_type=jnp.float32)
  wh1 = jnp.dot(h, w1_ref[...].T, preferred_element_type=jnp.float32)
  wh0_ref[...] = wh0
  wh1_ref[...] = wh1
  a0 = a0_ref[...]
  a1 = a1_ref[...]
  s0d = wh0 @ a0[0]
  s0s = wh0 @ a0[1]
  s1d = wh1 @ a1[0]
  s1s = wh1 @ a1[1]
  sd_ref[...] = jnp.stack([s0d, s1d], axis=0)
  ss_ref[...] = jnp.stack([s0s, s1s], axis=0)
  vals = jnp.concatenate([
      jnp.full((1, D), jnp.max(s0d), jnp.float32),
      jnp.full((1, D), jnp.max(s0s), jnp.float32),
      jnp.full((1, D), jnp.max(s1d), jnp.float32),
      jnp.full((1, D), jnp.max(s1s), jnp.float32),
      jnp.full((4, D), -3e38, jnp.float32),
  ], axis=0)
  j = pl.program_id(0)

  @pl.when(j == 0)
  def _():
    cm_ref[...] = vals

  @pl.when(j > 0)
  def _():
    cm_ref[...] = jnp.maximum(cm_ref[...], vals)


def _prep(h_pad, w0, w1, a0, a1):
  grid = NP // BN
  return pl.pallas_call(
      _prep_body,
      grid=(grid,),
      in_specs=[
          pl.BlockSpec((BN, D), lambda j: (j, 0)),
          pl.BlockSpec((D, D), lambda j: (0, 0)),
          pl.BlockSpec((D, D), lambda j: (0, 0)),
          pl.BlockSpec((2, D), lambda j: (0, 0)),
          pl.BlockSpec((2, D), lambda j: (0, 0)),
      ],
      out_specs=[
          pl.BlockSpec((BN, D), lambda j: (j, 0)),
          pl.BlockSpec((BN, D), lambda j: (j, 0)),
          pl.BlockSpec((2, BN), lambda j: (0, j)),
          pl.BlockSpec((2, BN), lambda j: (0, j)),
          pl.BlockSpec((8, D), lambda j: (0, 0)),
      ],
      out_shape=[
          jax.ShapeDtypeStruct((NP, D), jnp.float32),
          jax.ShapeDtypeStruct((NP, D), jnp.float32),
          jax.ShapeDtypeStruct((2, NP), jnp.float32),
          jax.ShapeDtypeStruct((2, NP), jnp.float32),
          jax.ShapeDtypeStruct((8, D), jnp.float32),
      ],
  )(h_pad, w0, w1, a0, a1)


def _sc_gat(wh0, wh1, sd0, ss0, sd1, ss1, row0, col0, row1, col1, cm):
  mesh = plsc.VectorSubcoreMesh(core_axis_name="c", subcore_axis_name="s")

  @functools.partial(
      pl.kernel,
      out_type=(
          jax.ShapeDtypeStruct((NP, D), jnp.float32),
          jax.ShapeDtypeStruct((NP, D), jnp.float32),
          jax.ShapeDtypeStruct((NSUB * NP,), jnp.float32),
          jax.ShapeDtypeStruct((NSUB * NP,), jnp.float32),
      ),
      mesh=mesh,
      compiler_params=pltpu.CompilerParams(needs_layout_passes=False),
      scratch_types=(
          pltpu.VMEM((N,), jnp.float32),
          pltpu.VMEM((N,), jnp.float32),
          pltpu.VMEM((KC,), jnp.int32),
          pltpu.VMEM((KC,), jnp.int32),
          pltpu.VMEM((KC, D), jnp.float32),
          pltpu.VMEM((KC,), jnp.int32),
          pltpu.VMEM((KC,), jnp.int32),
          pltpu.VMEM((KC,), jnp.int32),
          pltpu.VMEM((KC, D), jnp.float32),
          pltpu.VMEM((KC,), jnp.int32),
          pltpu.VMEM((16,), jnp.int32),
          pltpu.VMEM((16,), jnp.int32),
          pltpu.VMEM((16,), jnp.float32),
          pltpu.VMEM((16,), jnp.float32),
          pltpu.VMEM((N,), jnp.float32),
          pltpu.VMEM((RT,), jnp.float32),
          pltpu.VMEM((RT - 128,), jnp.float32),
          pltpu.VMEM_SHARED((NP, D), jnp.float32),
          pltpu.SemaphoreType.DMA,
          pltpu.SemaphoreType.DMA,
          pltpu.SemaphoreType.DMA,
          pltpu.SemaphoreType.DMA,
      ),
  )
  def k(wh0_h, wh1_h, sd0_h, ss0_h, sd1_h, ss1_h,
        row0_h, col0_h, row1_h, col1_h, cm_h,
        out0_h, out1_h, den0_h, den1_h,
        sd_v, ss_v, ir0, ic0, rg0, is0, ir1, ic1, rg1, is1,
        irt, ict, cd_v, cs_v, den_v, dtot_v, dpart_v, agg,
        sg0, sg1, si0, si1):
    c = lax.axis_index("c")
    s = lax.axis_index("s")
    zeros16 = jnp.zeros((16,), jnp.float32)
    bufs = ((ir0, ic0, rg0, is0, sg0, si0), (ir1, ic1, rg1, is1, sg1, si1))

    def phase1(sd_h, ss_h):
      pltpu.sync_copy(sd_h.at[pl.ds(0, N)], sd_v)
      pltpu.sync_copy(ss_h.at[pl.ds(0, N)], ss_v)

      def zrow(i, _):
        for b in range(D // 16):
          rg0[i, pl.ds(b * 16, 16)] = zeros16
        return 0
      lax.fori_loop(0, KC, zrow, 0)
      for t in range(NRB):
        pltpu.sync_copy(rg0, agg.at[pl.ds(s * RT + t * KC, KC), :])

      def zden(i, _):
        den_v[pl.ds(i * 16, 16)] = zeros16
        return 0
      lax.fori_loop(0, N // 16, zden, 0)

    def phase2(row_h, col_h, wh_h, den_h, crow):
      pltpu.sync_copy(cm_h.at[crow, pl.ds(0, 16)], cd_v)
      pltpu.sync_copy(cm_h.at[crow + 1, pl.ds(0, 16)], cs_v)
      cc = cd_v[...] + cs_v[...]
      tile_base = s * ET

      def start_idx(g, b):
        eb = tile_base + g * KC
        ir_b, ic_b, _, _, _, si_b = bufs[b]
        pltpu.async_copy(row_h.at[pl.ds(eb, KC)], ir_b, si_b)
        pltpu.async_copy(col_h.at[pl.ds(eb, KC)], ic_b, si_b)

      def wait_idx(b):
        ir_b, ic_b, _, _, _, si_b = bufs[b]
        pltpu.make_async_copy(row_h.at[pl.ds(0, KC)], ir_b, si_b).wait()
        pltpu.make_async_copy(col_h.at[pl.ds(0, KC)], ic_b, si_b).wait()

      def start_gather(b):
        _, ic_b, rg_b, _, sg_b, _ = bufs[b]
        pltpu.async_copy(wh_h.at[ic_b], rg_b, sg_b)

      def wait_gather(b):
        _, ic_b, rg_b, _, sg_b, _ = bufs[b]
        pltpu.make_async_copy(wh_h.at[ic_b], rg_b, sg_b).wait()

      def scatter(b):
        _, _, rg_b, is_b, _, _ = bufs[b]
        pltpu.sync_copy(rg_b, agg.at[is_b], add=True)


      def edge_block(ir_v, ic_v, rows_ref, j):
        ir = ir_v[pl.ds(j * 16, 16)]
        ic = ic_v[pl.ds(j * 16, 16)]
        e = plsc.load_gather(sd_v, [ir]) + plsc.load_gather(ss_v, [ic])
        e = jnp.where(e >= 0.0, e, e * LEAKY)
        p = jnp.exp(e - cc)
        plsc.addupdate_scatter(den_v, [ir], p)
        for lane in range(16):
          i = j * 16 + lane
          p_i = p[lane]
          for bb in range(D // 16):
            rows_ref[i, pl.ds(bb * 16, 16)] = (
                rows_ref[i, pl.ds(bb * 16, 16)] * p_i)

      def compute(b):
        ir_b, ic_b, rg_b, is_b, _, _ = bufs[b]

        def rblk(j, _):
          edge_block(ir_b, ic_b, rg_b, j)
          return 0
        lax.fori_loop(0, KC // 16, rblk, 0)
        for j in range(KC // 16):
          is_b[pl.ds(j * 16, 16)] = ir_b[pl.ds(j * 16, 16)]

      start_idx(0, 0)
      wait_idx(0)
      start_gather(0)
      start_idx(1, 1)

      def pair(t, _):
        wait_idx(1)
        start_gather(1)
        wait_gather(0)
        compute(0)

        @pl.when(t < NPAIR - 1)
        def _():
          start_idx(2 * t + 2, 0)
        scatter(0)

        @pl.when(t < NPAIR - 1)
        def _():
          wait_idx(0)
          start_gather(0)
        wait_gather(1)
        compute(1)

        @pl.when(t < NPAIR - 1)
        def _():
          start_idx(2 * t + 3, 1)
        scatter(1)
        return 0
      lax.fori_loop(0, NPAIR, pair, 0)

      tb = tile_base + NFULL * KC
      pltpu.sync_copy(row_h.at[pl.ds(tb, TAIL)], irt)
      pltpu.sync_copy(col_h.at[pl.ds(tb, TAIL)], ict)
      pltpu.async_copy(wh_h.at[ict], rg0.at[pl.ds(0, TAIL), :], sg0).wait()
      edge_block(irt, ict, rg0, 0)
      pltpu.sync_copy(rg0.at[pl.ds(0, TAIL), :], agg.at[irt], add=True)

      pltpu.sync_copy(den_v, den_h.at[pl.ds(s * NP, N)])

    def phase3(out_h, den_h):
      pltpu.sync_copy(den_h.at[pl.ds(s * RT, RT)], dtot_v)
      for t in range(1, NSUB):
        for (off, ln) in ((0, RT - 128), (RT - 128, 128)):
          pltpu.sync_copy(den_h.at[pl.ds(t * NP + s * RT + off, ln)],
                          dpart_v.at[pl.ds(0, ln)])

          def dacc(i, _):
            dtot_v[pl.ds(off + i * 16, 16)] = (
                dtot_v[pl.ds(off + i * 16, 16)] + dpart_v[pl.ds(i * 16, 16)])
            return 0
          lax.fori_loop(0, ln // 16, dacc, 0)

      def dinv(i, _):
        d = dtot_v[pl.ds(i * 16, 16)]
        dtot_v[pl.ds(i * 16, 16)] = 1.0 / jnp.maximum(d, 1e-12)
        return 0
      lax.fori_loop(0, RT // 16, dinv, 0)

      def blk(t, _):
        base = s * RT + t * KC
        pltpu.sync_copy(agg.at[pl.ds(base, KC), :], rg0)

        def nblk(j, _):
          iv16 = dtot_v[pl.ds(t * KC + j * 16, 16)]
          for lane in range(16):
            i = j * 16 + lane
            inv = iv16[lane]
            for b in range(D // 16):
              rg0[i, pl.ds(b * 16, 16)] = (
                  rg0[i, pl.ds(b * 16, 16)] * inv)
          return 0
        lax.fori_loop(0, KC // 16, nblk, 0)
        pltpu.sync_copy(rg0, out_h.at[pl.ds(base, KC), :])
        return 0
      lax.fori_loop(0, NRB, blk, 0)

    pl.when(c == 0)(lambda: phase1(sd0_h, ss0_h))
    pl.when(c == 1)(lambda: phase1(sd1_h, ss1_h))
    plsc.subcore_barrier()
    pl.when(c == 0)(lambda: phase2(row0_h, col0_h, wh0_h, den0_h, 0))
    pl.when(c == 1)(lambda: phase2(row1_h, col1_h, wh1_h, den1_h, 2))
    plsc.subcore_barrier()
    pl.when(c == 0)(lambda: phase3(out0_h, den0_h))
    pl.when(c == 1)(lambda: phase3(out1_h, den1_h))

  return k(wh0, wh1, sd0, ss0, sd1, ss1, row0, col0, row1, col1, cm)


def _combine_body(c0_ref, c1_ref, b_ref, o_ref):
  o_ref[...] = c0_ref[...] + c1_ref[...] + b_ref[...]


def _combine(c0, c1, bias2d):
  grid = NP // BN
  return pl.pallas_call(
      _combine_body,
      grid=(grid,),
      in_specs=[
          pl.BlockSpec((BN, D), lambda j: (j, 0)),
          pl.BlockSpec((BN, D), lambda j: (j, 0)),
          pl.BlockSpec((1, D), lambda j: (0, 0)),
      ],
      out_specs=pl.BlockSpec((BN, D), lambda j: (j, 0)),
      out_shape=jax.ShapeDtypeStruct((NP, D), jnp.float32),
  )(c0, c1, bias2d)


@jax.jit
def kernel(H, W_r0, W_r1, a_r0, a_r1, bias, row_r0, col_r0, row_r1, col_r1):
  h_pad = jnp.pad(H, ((0, NP - N), (0, 0)))
  a0 = a_r0.reshape(2, D)
  a1 = a_r1.reshape(2, D)
  wh0, wh1, sd, ss, cm = _prep(h_pad, W_r0, W_r1, a0, a1)
  c0, c1, _, _ = _sc_gat(
      wh0, wh1, sd[0], ss[0], sd[1], ss[1],
      row_r0.astype(jnp.int32), col_r0.astype(jnp.int32),
      row_r1.astype(jnp.int32), col_r1.astype(jnp.int32), cm,
  )
  out = _combine(c0, c1, bias.reshape(1, D))
  return out[:N]
